# baseline (speedup 1.0000x reference)
import jax
import jax.numpy as jnp
from jax.experimental import pallas as pl
from jax.experimental.pallas import tpu as pltpu

_BETA = 0.5
_ALPHA = 0.2


def _gat_block_kernel(inp_ref, inp_blk_ref, w_ref, a_ref, adj_ref, m_ref, out_ref):
    h = jnp.dot(inp_ref[:], w_ref[:], preferred_element_type=jnp.float32)
    dout = h.shape[1]
    a1 = a_ref[0, :dout]
    a2 = a_ref[0, dout:]
    g = jnp.dot(h, a2, preferred_element_type=jnp.float32)
    h_blk = jnp.dot(inp_blk_ref[:], w_ref[:], preferred_element_type=jnp.float32)
    f_blk = jnp.dot(h_blk, a1, preferred_element_type=jnp.float32)

    adj_blk = adj_ref[:].astype(jnp.float32)
    mb = _BETA * m_ref[:] + (1.0 - _BETA) * adj_blk
    logit = f_blk[:, None] + mb * g[None, :]
    e = adj_blk * jnp.exp(jnp.where(logit >= 0, logit, _ALPHA * logit))
    rowsum = jnp.sum(e, axis=1, keepdims=True)
    hp = jnp.dot(e, h, preferred_element_type=jnp.float32) / rowsum
    out_ref[:] = jnp.where(hp > 0, hp, jnp.exp(jnp.minimum(hp, 0.0)) - 1.0)


def kernel(input, adj, M, W, a):
    N, din = input.shape
    dout = W.shape[1]
    block_rows = 512
    grid = (N // block_rows,)
    adj8 = adj.astype(jnp.int8)
    return pl.pallas_call(
        _gat_block_kernel,
        grid=grid,
        in_specs=[
            pl.BlockSpec((N, din), lambda i: (0, 0)),
            pl.BlockSpec((block_rows, din), lambda i: (i, 0)),
            pl.BlockSpec((din, dout), lambda i: (0, 0)),
            pl.BlockSpec((1, 2 * dout), lambda i: (0, 0)),
            pl.BlockSpec((block_rows, N), lambda i: (i, 0)),
            pl.BlockSpec((block_rows, N), lambda i: (i, 0)),
        ],
        out_specs=pl.BlockSpec((block_rows, dout), lambda i: (i, 0)),
        out_shape=jax.ShapeDtypeStruct((N, dout), jnp.float32),
        compiler_params=pltpu.CompilerParams(
            dimension_semantics=("arbitrary",),
        ),
    )(input, input, W, a, adj8, M)


# speedup vs baseline: 1.2560x; 1.2560x over previous
import jax
import jax.numpy as jnp
from jax.experimental import pallas as pl
from jax.experimental.pallas import tpu as pltpu

_BETA = 0.5
_ALPHA = 0.2


def _gat_block_kernel(inp_ref, inp_blk_ref, w_ref, a_ref, adj_ref, m_ref, out_ref):
    h = jnp.dot(inp_ref[:], w_ref[:], preferred_element_type=jnp.float32)
    dout = h.shape[1]
    a1 = a_ref[0, :dout]
    a2 = a_ref[0, dout:]
    g = jnp.dot(h, a2, preferred_element_type=jnp.float32)
    h_blk = jnp.dot(inp_blk_ref[:], w_ref[:], preferred_element_type=jnp.float32)
    f_blk = jnp.dot(h_blk, a1, preferred_element_type=jnp.float32)

    adj_blk = adj_ref[:]
    mb = _BETA * m_ref[:] + (1.0 - _BETA) * adj_blk
    logit = f_blk[:, None] + mb * g[None, :]
    e = adj_blk * jnp.exp(jnp.where(logit >= 0, logit, _ALPHA * logit))
    rowsum = jnp.sum(e, axis=1, keepdims=True)
    hp = jnp.dot(e, h, preferred_element_type=jnp.float32) / rowsum
    out_ref[:] = jnp.where(hp > 0, hp, jnp.exp(jnp.minimum(hp, 0.0)) - 1.0)


def kernel(input, adj, M, W, a):
    N, din = input.shape
    dout = W.shape[1]
    block_rows = 256
    grid = (N // block_rows,)
    return pl.pallas_call(
        _gat_block_kernel,
        grid=grid,
        in_specs=[
            pl.BlockSpec((N, din), lambda i: (0, 0)),
            pl.BlockSpec((block_rows, din), lambda i: (i, 0)),
            pl.BlockSpec((din, dout), lambda i: (0, 0)),
            pl.BlockSpec((1, 2 * dout), lambda i: (0, 0)),
            pl.BlockSpec((block_rows, N), lambda i: (i, 0)),
            pl.BlockSpec((block_rows, N), lambda i: (i, 0)),
        ],
        out_specs=pl.BlockSpec((block_rows, dout), lambda i: (i, 0)),
        out_shape=jax.ShapeDtypeStruct((N, dout), jnp.float32),
        compiler_params=pltpu.CompilerParams(
            dimension_semantics=("parallel",),
        ),
    )(input, input, W, a, adj, M)


# speedup vs baseline: 1.3788x; 1.0978x over previous
import jax
import jax.numpy as jnp
from jax.experimental import pallas as pl
from jax.experimental.pallas import tpu as pltpu

_BETA = 0.5
_ALPHA = 0.2


def _gat_block_kernel(inp_ref, inp_blk_ref, w_ref, a_ref, adj_ref, m_ref, out_ref):
    h = jnp.dot(inp_ref[:], w_ref[:], preferred_element_type=jnp.float32)
    dout = h.shape[1]
    a1 = a_ref[0, :dout]
    a2 = a_ref[0, dout:]
    g = jnp.dot(h, a2, preferred_element_type=jnp.float32)
    h_blk = jnp.dot(inp_blk_ref[:], w_ref[:], preferred_element_type=jnp.float32)
    f_blk = jnp.dot(h_blk, a1, preferred_element_type=jnp.float32)

    adj_blk = adj_ref[:]
    mb = _BETA * m_ref[:] + (1.0 - _BETA) * adj_blk
    logit = f_blk[:, None] + mb * g[None, :]
    e = adj_blk * jnp.exp(jnp.where(logit >= 0, logit, _ALPHA * logit))
    rowsum = jnp.sum(e, axis=1, keepdims=True)
    hp = jnp.dot(e, h, preferred_element_type=jnp.float32) / rowsum
    out_ref[:] = jnp.where(hp > 0, hp, jnp.exp(jnp.minimum(hp, 0.0)) - 1.0)


def kernel(input, adj, M, W, a):
    N, din = input.shape
    dout = W.shape[1]
    block_rows = 512
    grid = (N // block_rows,)
    return pl.pallas_call(
        _gat_block_kernel,
        grid=grid,
        in_specs=[
            pl.BlockSpec((N, din), lambda i: (0, 0)),
            pl.BlockSpec((block_rows, din), lambda i: (i, 0)),
            pl.BlockSpec((din, dout), lambda i: (0, 0)),
            pl.BlockSpec((1, 2 * dout), lambda i: (0, 0)),
            pl.BlockSpec((block_rows, N), lambda i: (i, 0)),
            pl.BlockSpec((block_rows, N), lambda i: (i, 0)),
        ],
        out_specs=pl.BlockSpec((block_rows, dout), lambda i: (i, 0)),
        out_shape=jax.ShapeDtypeStruct((N, dout), jnp.float32),
        compiler_params=pltpu.CompilerParams(
            dimension_semantics=("parallel",),
        ),
    )(input, input, W, a, adj, M)


# speedup vs baseline: 1.3852x; 1.0047x over previous
import jax
import jax.numpy as jnp
from jax.experimental import pallas as pl
from jax.experimental.pallas import tpu as pltpu

_BETA = 0.5
_ALPHA = 0.2


def _gat_block_kernel(inp_ref, inp_blk_ref, w_ref, a_ref,
                      adj_l_ref, adj_r_ref, m_l_ref, m_r_ref, out_ref):
    h = jnp.dot(inp_ref[:], w_ref[:], preferred_element_type=jnp.float32)
    n = h.shape[0]
    dout = h.shape[1]
    a1 = a_ref[0, :dout]
    a2 = a_ref[0, dout:]
    g = jnp.dot(h, a2, preferred_element_type=jnp.float32)
    h_blk = jnp.dot(inp_blk_ref[:], w_ref[:], preferred_element_type=jnp.float32)
    f_blk = jnp.dot(h_blk, a1, preferred_element_type=jnp.float32)[:, None]

    half = n // 2

    def att(adj_c, m_c, g_c):
        mb = _BETA * m_c + (1.0 - _BETA) * adj_c
        logit = f_blk + mb * g_c[None, :]
        return adj_c * jnp.exp(jnp.where(logit >= 0, logit, _ALPHA * logit))

    e_l = att(adj_l_ref[:], m_l_ref[:], g[:half])
    e_r = att(adj_r_ref[:], m_r_ref[:], g[half:])
    rowsum = (jnp.sum(e_l, axis=1) + jnp.sum(e_r, axis=1))[:, None]
    hp = (jnp.dot(e_l, h[:half], preferred_element_type=jnp.float32)
          + jnp.dot(e_r, h[half:], preferred_element_type=jnp.float32)) / rowsum
    out_ref[:] = jnp.where(hp > 0, hp, jnp.exp(jnp.minimum(hp, 0.0)) - 1.0)


def kernel(input, adj, M, W, a):
    N, din = input.shape
    dout = W.shape[1]
    block_rows = 512
    half = N // 2
    grid = (N // block_rows,)
    return pl.pallas_call(
        _gat_block_kernel,
        grid=grid,
        in_specs=[
            pl.BlockSpec((N, din), lambda i: (0, 0)),
            pl.BlockSpec((block_rows, din), lambda i: (i, 0)),
            pl.BlockSpec((din, dout), lambda i: (0, 0)),
            pl.BlockSpec((1, 2 * dout), lambda i: (0, 0)),
            pl.BlockSpec((block_rows, half), lambda i: (i, 0)),
            pl.BlockSpec((block_rows, half), lambda i: (i, 1)),
            pl.BlockSpec((block_rows, half), lambda i: (i, 0)),
            pl.BlockSpec((block_rows, half), lambda i: (i, 1)),
        ],
        out_specs=pl.BlockSpec((block_rows, dout), lambda i: (i, 0)),
        out_shape=jax.ShapeDtypeStruct((N, dout), jnp.float32),
        compiler_params=pltpu.CompilerParams(
            dimension_semantics=("parallel",),
        ),
    )(input, input, W, a, adj, adj, M, M)
